# baseline (speedup 1.0000x reference)
import jax
import jax.numpy as jnp
from jax import lax
from jax.experimental import pallas as pl
from jax.experimental.pallas import tpu as pltpu
from jax.experimental.pallas import tpu_sc as plsc

_NUM_USERS = 6000
_NUM_ITEMS = 4000
_N = _NUM_USERS + _NUM_ITEMS
_D = 128
_LAYERS = 3
_E = 320000
_BATCH = 4096

_NC = 2
_NS = 16
_NW = _NC * _NS

_EB = 50
_EPW = _E // _NW
_NBLK = _EPW // _EB
_CH = 40
_NCHUNK = _NBLK // _CH
_RPT = 624
_GPT = _BATCH // _NW

_mesh = plsc.VectorSubcoreMesh(core_axis_name="c", subcore_axis_name="s",
                               num_cores=_NC, num_subcores=_NS)


def _propagate_body(curv_hbm, rows_hbm, cols_hbm, vals_hbm, out_hbm,
                    rows_v, cols_v, vals_v, gb0, gb1, gb2, gb3,
                    sb0, sb1, accum,
                    semg0, semg1, semg2, semg3, sems0, sems1):
    cid = lax.axis_index("c")
    sid = lax.axis_index("s")
    wid = sid * _NC + cid

    z16 = jnp.zeros((16,), jnp.float32)

    def zrow(i, carry):
        for d in range(_D // 16):
            sb0[i, pl.ds(d * 16, 16)] = z16
        return carry

    lax.fori_loop(0, 48, zrow, 0)
    zd = [pltpu.async_copy(sb0.at[pl.ds(0, 48)],
                           accum.at[pl.ds(sid * _RPT + k * 48, 48)], semg0)
          for k in range(_RPT // 48)]
    for d in zd:
        d.wait()

    @pl.when(sid == 0)
    def _():
        pltpu.sync_copy(sb0.at[pl.ds(0, 16)], accum.at[pl.ds(_NS * _RPT, 16)])

    plsc.subcore_barrier()

    gb = (gb0, gb1, gb2, gb3)
    sb = (sb0, sb1)
    sg = (semg0, semg1, semg2, semg3)
    ss = (sems0, sems1)

    def gather_wait(bb, rg):
        pltpu.make_async_copy(curv_hbm.at[cols_v.at[bb]], gb[rg], sg[rg]).wait()

    def scatter_wait(bb, r):
        pltpu.make_async_copy(sb[r], accum.at[rows_v.at[bb]], ss[r]).wait()

    def scale(bb, rg, rs):
        @plsc.parallel_loop(0, _EB, 1, unroll=10)
        def _(j):
            v = plsc.load_gather(
                vals_v, [jnp.zeros((16,), jnp.int32) + (bb * _EB + j)])
            for q in range(_D // 32):
                w = gb[rg][j, pl.ds(q * 16, 16)]
                pair = plsc.bitcast(w, jnp.bfloat16)
                lo, hi = plsc.unpack(pair, format=plsc.PackFormat.INTERLEAVED)
                sb[rs][j, pl.ds(q * 32, 16)] = lo * v
                sb[rs][j, pl.ds(q * 32 + 16, 16)] = hi * v

    for c in range(_NCHUNK):
        sd = [pltpu.async_copy(rows_hbm.at[wid, pl.ds(c * _CH, _CH)],
                               rows_v, semg0),
              pltpu.async_copy(cols_hbm.at[wid, pl.ds(c * _CH, _CH)],
                               cols_v, semg1),
              pltpu.async_copy(
                  vals_hbm.at[pl.ds(wid * _EPW + c * _CH * _EB, _CH * _EB)],
                  vals_v, semg2)]
        for d in sd:
            d.wait()

        for k in range(4):
            pltpu.async_copy(curv_hbm.at[cols_v.at[k]], gb[k], sg[k])

        def quad(g, carry):
            for rg in range(4):
                b = 4 * g + rg
                rs = rg % 2
                gather_wait(b, rg)

                @pl.when(b >= 2)
                def _():
                    scatter_wait(b - 2, rs)

                scale(b, rg, rs)

                @pl.when(b <= _CH - 5)
                def _():
                    pltpu.async_copy(
                        curv_hbm.at[cols_v.at[b + 4]], gb[rg], sg[rg])

                pltpu.async_copy(sb[rs], accum.at[rows_v.at[b]], ss[rs],
                                 add=True)
            return carry

        lax.fori_loop(0, _CH // 4, quad, 0)
        scatter_wait(_CH - 2, 0)
        scatter_wait(_CH - 1, 1)

    plsc.subcore_barrier()

    pltpu.sync_copy(accum.at[pl.ds(sid * _RPT, _RPT)],
                    out_hbm.at[cid, pl.ds(sid * _RPT, _RPT)])

    @pl.when(sid == 0)
    def _():
        pltpu.sync_copy(accum.at[pl.ds(_NS * _RPT, 16)],
                        out_hbm.at[cid, pl.ds(_NS * _RPT, 16)])


_propagate = pl.kernel(
    _propagate_body,
    out_type=jax.ShapeDtypeStruct((_NC, _N, _D), jnp.float32),
    mesh=_mesh,
    compiler_params=pltpu.CompilerParams(needs_layout_passes=False,
                                         use_tc_tiling_on_sc=False),
    scratch_types=[
        pltpu.VMEM((_CH, _EB), jnp.int32),
        pltpu.VMEM((_CH, _EB), jnp.int32),
        pltpu.VMEM((_CH * _EB,), jnp.float32),
        pltpu.VMEM((_EB, _D // 2), jnp.float32),
        pltpu.VMEM((_EB, _D // 2), jnp.float32),
        pltpu.VMEM((_EB, _D // 2), jnp.float32),
        pltpu.VMEM((_EB, _D // 2), jnp.float32),
        pltpu.VMEM((_EB, _D), jnp.float32),
        pltpu.VMEM((_EB, _D), jnp.float32),
        pltpu.VMEM_SHARED((_N, _D), jnp.float32),
        pltpu.SemaphoreType.DMA,
        pltpu.SemaphoreType.DMA,
        pltpu.SemaphoreType.DMA,
        pltpu.SemaphoreType.DMA,
        pltpu.SemaphoreType.DMA,
        pltpu.SemaphoreType.DMA,
    ],
)


def _norm_body(p_ref, o_ref):
    x = p_ref[0] + p_ref[1]
    ss = jnp.sum(x * x, axis=1, keepdims=True)
    o_ref[...] = x / jnp.maximum(jnp.sqrt(ss), 1e-12)


_norm = pl.pallas_call(
    _norm_body,
    grid=(10,),
    in_specs=[pl.BlockSpec((_NC, _N // 10, _D), lambda i: (0, i, 0))],
    out_specs=pl.BlockSpec((_N // 10, _D), lambda i: (i, 0)),
    out_shape=jax.ShapeDtypeStruct((_N, _D), jnp.float32),
)


def _final_body(p_ref, a_ref, b_ref, c_ref, o_ref):
    x = p_ref[0] + p_ref[1]
    ss = jnp.sum(x * x, axis=1, keepdims=True)
    x = x / jnp.maximum(jnp.sqrt(ss), 1e-12)
    o_ref[...] = 0.25 * (a_ref[...] + b_ref[...] + c_ref[...] + x)


_final = pl.pallas_call(
    _final_body,
    grid=(10,),
    in_specs=[
        pl.BlockSpec((_NC, _N // 10, _D), lambda i: (0, i, 0)),
        pl.BlockSpec((_N // 10, _D), lambda i: (i, 0)),
        pl.BlockSpec((_N // 10, _D), lambda i: (i, 0)),
        pl.BlockSpec((_N // 10, _D), lambda i: (i, 0)),
    ],
    out_specs=pl.BlockSpec((_N // 10, _D), lambda i: (i, 0)),
    out_shape=jax.ShapeDtypeStruct((_N, _D), jnp.float32),
)


def _gather_body(final_hbm, uid_hbm, pid_hbm, nid_hbm,
                 ug_hbm, pg_hbm, ng_hbm,
                 uidx, pidx, nidx, ubuf, pbuf, nbuf, sem):
    wid = lax.axis_index("s") * _NC + lax.axis_index("c")
    base = wid * _GPT
    sd = [pltpu.async_copy(uid_hbm.at[pl.ds(base, _GPT)], uidx, sem),
          pltpu.async_copy(pid_hbm.at[pl.ds(base, _GPT)], pidx, sem),
          pltpu.async_copy(nid_hbm.at[pl.ds(base, _GPT)], nidx, sem)]
    for d in sd:
        d.wait()
    off = jnp.full((16,), _NUM_USERS, jnp.int32)
    for k in range(_GPT // 16):
        sl = pl.ds(k * 16, 16)
        pidx[sl] = pidx[sl] + off
        nidx[sl] = nidx[sl] + off
    gd = [pltpu.async_copy(final_hbm.at[uidx], ubuf, sem),
          pltpu.async_copy(final_hbm.at[pidx], pbuf, sem),
          pltpu.async_copy(final_hbm.at[nidx], nbuf, sem)]
    for d in gd:
        d.wait()
    wd = [pltpu.async_copy(ubuf, ug_hbm.at[pl.ds(base, _GPT)], sem),
          pltpu.async_copy(pbuf, pg_hbm.at[pl.ds(base, _GPT)], sem),
          pltpu.async_copy(nbuf, ng_hbm.at[pl.ds(base, _GPT)], sem)]
    for d in wd:
        d.wait()


_gather = pl.kernel(
    _gather_body,
    compiler_params=pltpu.CompilerParams(needs_layout_passes=False),
    out_type=(
        jax.ShapeDtypeStruct((_BATCH, _D), jnp.float32),
        jax.ShapeDtypeStruct((_BATCH, _D), jnp.float32),
        jax.ShapeDtypeStruct((_BATCH, _D), jnp.float32),
    ),
    mesh=_mesh,
    scratch_types=[
        pltpu.VMEM((_GPT,), jnp.int32),
        pltpu.VMEM((_GPT,), jnp.int32),
        pltpu.VMEM((_GPT,), jnp.int32),
        pltpu.VMEM((_GPT, _D), jnp.float32),
        pltpu.VMEM((_GPT, _D), jnp.float32),
        pltpu.VMEM((_GPT, _D), jnp.float32),
        pltpu.SemaphoreType.DMA,
    ],
)


def _loss_body(u_ref, p_ref, n_ref, o_ref):
    u = u_ref[...]
    diff = jnp.sum(u * (p_ref[...] - n_ref[...]), axis=1)
    o_ref[...] = jnp.reshape(-jnp.mean(jax.nn.log_sigmoid(diff)), (1, 1))


_loss = pl.pallas_call(
    _loss_body,
    out_shape=jax.ShapeDtypeStruct((1, 1), jnp.float32),
)


def _gather_view(x):
    xb = x.astype(jnp.bfloat16).reshape(-1, 4, 2, 16)
    xb = jnp.transpose(xb, (0, 1, 3, 2)).reshape(-1, 64, 2)
    return jax.lax.bitcast_convert_type(xb, jnp.float32)


def kernel(user_id, pos_item, neg_item, edge_index, edge_values, user_emb, item_emb):
    rows = edge_index[0].astype(jnp.int32).reshape(_NW, _NBLK, _EB)
    cols = edge_index[1].astype(jnp.int32).reshape(_NW, _NBLK, _EB)
    vals = edge_values

    e0 = jnp.concatenate([user_emb, item_emb], axis=0)

    cur = e0
    normed = []
    for layer in range(_LAYERS - 1):
        parts = _propagate(_gather_view(cur), rows, cols, vals)
        cur = _norm(parts)
        normed.append(cur)
    parts = _propagate(_gather_view(cur), rows, cols, vals)
    final_embedding = _final(parts, e0, normed[0], normed[1])

    ug, pg, ng = _gather(final_embedding,
                         user_id.astype(jnp.int32),
                         pos_item.astype(jnp.int32),
                         neg_item.astype(jnp.int32))
    rec_loss = _loss(ug, pg, ng)[0, 0]
    return (rec_loss, final_embedding)


# speedup vs baseline: 1.0070x; 1.0070x over previous
import jax
import jax.numpy as jnp
from jax import lax
from jax.experimental import pallas as pl
from jax.experimental.pallas import tpu as pltpu
from jax.experimental.pallas import tpu_sc as plsc

_NUM_USERS = 6000
_NUM_ITEMS = 4000
_N = _NUM_USERS + _NUM_ITEMS
_D = 128
_LAYERS = 3
_E = 320000
_BATCH = 4096

_NC = 2
_NS = 16
_NW = _NC * _NS

_EB = 50
_EPW = _E // _NW
_NBLK = _EPW // _EB
_CH = 40
_NCHUNK = _NBLK // _CH
_RPT = 624
_GPT = _BATCH // _NW

_mesh = plsc.VectorSubcoreMesh(core_axis_name="c", subcore_axis_name="s",
                               num_cores=_NC, num_subcores=_NS)


def _propagate_body(curv_hbm, rows_hbm, cols_hbm, vals_hbm, out_hbm,
                    rows_v, cols_v, vals_v, gb0, gb1, gb2, gb3,
                    sb0, sb1, accum,
                    semg0, semg1, semg2, semg3, sems0, sems1):
    cid = lax.axis_index("c")
    sid = lax.axis_index("s")
    wid = sid * _NC + cid

    z16 = jnp.zeros((16,), jnp.float32)

    def zrow(i, carry):
        for d in range(_D // 16):
            sb0[i, pl.ds(d * 16, 16)] = z16
        return carry

    lax.fori_loop(0, 48, zrow, 0)
    zd = [pltpu.async_copy(sb0.at[pl.ds(0, 48)],
                           accum.at[pl.ds(sid * _RPT + k * 48, 48)], semg0)
          for k in range(_RPT // 48)]
    for d in zd:
        d.wait()

    @pl.when(sid == 0)
    def _():
        pltpu.sync_copy(sb0.at[pl.ds(0, 16)], accum.at[pl.ds(_NS * _RPT, 16)])

    plsc.subcore_barrier()

    gb = (gb0, gb1, gb2, gb3)
    sb = (sb0, sb1)
    sg = (semg0, semg1, semg2, semg3)
    ss = (sems0, sems1)

    def gather_wait(bb, rg):
        pltpu.make_async_copy(curv_hbm.at[cols_v.at[bb]], gb[rg], sg[rg]).wait()

    def scatter_wait(bb, r):
        pltpu.make_async_copy(sb[r], accum.at[rows_v.at[bb]], ss[r]).wait()

    def scale(bb, rg, rs):
        @plsc.parallel_loop(0, _EB, 1, unroll=5)
        def _(j):
            v = plsc.load_gather(
                vals_v, [jnp.zeros((16,), jnp.int32) + (bb * _EB + j)])
            for q in range(_D // 32):
                w = gb[rg][j, pl.ds(q * 16, 16)]
                pair = plsc.bitcast(w, jnp.bfloat16)
                lo, hi = plsc.unpack(pair, format=plsc.PackFormat.INTERLEAVED)
                sb[rs][j, pl.ds(q * 32, 16)] = lo * v
                sb[rs][j, pl.ds(q * 32 + 16, 16)] = hi * v

    for c in range(_NCHUNK):
        sd = [pltpu.async_copy(rows_hbm.at[wid, pl.ds(c * _CH, _CH)],
                               rows_v, semg0),
              pltpu.async_copy(cols_hbm.at[wid, pl.ds(c * _CH, _CH)],
                               cols_v, semg1),
              pltpu.async_copy(
                  vals_hbm.at[pl.ds(wid * _EPW + c * _CH * _EB, _CH * _EB)],
                  vals_v, semg2)]
        for d in sd:
            d.wait()

        for k in range(4):
            pltpu.async_copy(curv_hbm.at[cols_v.at[k]], gb[k], sg[k])

        def quad(g, carry):
            for rg in range(4):
                b = 4 * g + rg
                rs = rg % 2
                gather_wait(b, rg)

                @pl.when(b >= 2)
                def _():
                    scatter_wait(b - 2, rs)

                scale(b, rg, rs)

                @pl.when(b <= _CH - 5)
                def _():
                    pltpu.async_copy(
                        curv_hbm.at[cols_v.at[b + 4]], gb[rg], sg[rg])

                pltpu.async_copy(sb[rs], accum.at[rows_v.at[b]], ss[rs],
                                 add=True)
            return carry

        lax.fori_loop(0, _CH // 4, quad, 0)
        scatter_wait(_CH - 2, 0)
        scatter_wait(_CH - 1, 1)

    plsc.subcore_barrier()

    pltpu.sync_copy(accum.at[pl.ds(sid * _RPT, _RPT)],
                    out_hbm.at[cid, pl.ds(sid * _RPT, _RPT)])

    @pl.when(sid == 0)
    def _():
        pltpu.sync_copy(accum.at[pl.ds(_NS * _RPT, 16)],
                        out_hbm.at[cid, pl.ds(_NS * _RPT, 16)])


_propagate = pl.kernel(
    _propagate_body,
    out_type=jax.ShapeDtypeStruct((_NC, _N, _D), jnp.float32),
    mesh=_mesh,
    compiler_params=pltpu.CompilerParams(needs_layout_passes=False,
                                         use_tc_tiling_on_sc=False),
    scratch_types=[
        pltpu.VMEM((_CH, _EB), jnp.int32),
        pltpu.VMEM((_CH, _EB), jnp.int32),
        pltpu.VMEM((_CH * _EB,), jnp.float32),
        pltpu.VMEM((_EB, _D // 2), jnp.float32),
        pltpu.VMEM((_EB, _D // 2), jnp.float32),
        pltpu.VMEM((_EB, _D // 2), jnp.float32),
        pltpu.VMEM((_EB, _D // 2), jnp.float32),
        pltpu.VMEM((_EB, _D), jnp.float32),
        pltpu.VMEM((_EB, _D), jnp.float32),
        pltpu.VMEM_SHARED((_N, _D), jnp.float32),
        pltpu.SemaphoreType.DMA,
        pltpu.SemaphoreType.DMA,
        pltpu.SemaphoreType.DMA,
        pltpu.SemaphoreType.DMA,
        pltpu.SemaphoreType.DMA,
        pltpu.SemaphoreType.DMA,
    ],
)


def _norm_body(p_ref, o_ref):
    x = p_ref[0] + p_ref[1]
    ss = jnp.sum(x * x, axis=1, keepdims=True)
    o_ref[...] = x / jnp.maximum(jnp.sqrt(ss), 1e-12)


_norm = pl.pallas_call(
    _norm_body,
    grid=(10,),
    in_specs=[pl.BlockSpec((_NC, _N // 10, _D), lambda i: (0, i, 0))],
    out_specs=pl.BlockSpec((_N // 10, _D), lambda i: (i, 0)),
    out_shape=jax.ShapeDtypeStruct((_N, _D), jnp.float32),
)


def _final_body(p_ref, a_ref, b_ref, c_ref, o_ref):
    x = p_ref[0] + p_ref[1]
    ss = jnp.sum(x * x, axis=1, keepdims=True)
    x = x / jnp.maximum(jnp.sqrt(ss), 1e-12)
    o_ref[...] = 0.25 * (a_ref[...] + b_ref[...] + c_ref[...] + x)


_final = pl.pallas_call(
    _final_body,
    grid=(10,),
    in_specs=[
        pl.BlockSpec((_NC, _N // 10, _D), lambda i: (0, i, 0)),
        pl.BlockSpec((_N // 10, _D), lambda i: (i, 0)),
        pl.BlockSpec((_N // 10, _D), lambda i: (i, 0)),
        pl.BlockSpec((_N // 10, _D), lambda i: (i, 0)),
    ],
    out_specs=pl.BlockSpec((_N // 10, _D), lambda i: (i, 0)),
    out_shape=jax.ShapeDtypeStruct((_N, _D), jnp.float32),
)


def _gather_body(final_hbm, uid_hbm, pid_hbm, nid_hbm,
                 ug_hbm, pg_hbm, ng_hbm,
                 uidx, pidx, nidx, ubuf, pbuf, nbuf, sem):
    wid = lax.axis_index("s") * _NC + lax.axis_index("c")
    base = wid * _GPT
    sd = [pltpu.async_copy(uid_hbm.at[pl.ds(base, _GPT)], uidx, sem),
          pltpu.async_copy(pid_hbm.at[pl.ds(base, _GPT)], pidx, sem),
          pltpu.async_copy(nid_hbm.at[pl.ds(base, _GPT)], nidx, sem)]
    for d in sd:
        d.wait()
    off = jnp.full((16,), _NUM_USERS, jnp.int32)
    for k in range(_GPT // 16):
        sl = pl.ds(k * 16, 16)
        pidx[sl] = pidx[sl] + off
        nidx[sl] = nidx[sl] + off
    gd = [pltpu.async_copy(final_hbm.at[uidx], ubuf, sem),
          pltpu.async_copy(final_hbm.at[pidx], pbuf, sem),
          pltpu.async_copy(final_hbm.at[nidx], nbuf, sem)]
    for d in gd:
        d.wait()
    wd = [pltpu.async_copy(ubuf, ug_hbm.at[pl.ds(base, _GPT)], sem),
          pltpu.async_copy(pbuf, pg_hbm.at[pl.ds(base, _GPT)], sem),
          pltpu.async_copy(nbuf, ng_hbm.at[pl.ds(base, _GPT)], sem)]
    for d in wd:
        d.wait()


_gather = pl.kernel(
    _gather_body,
    compiler_params=pltpu.CompilerParams(needs_layout_passes=False),
    out_type=(
        jax.ShapeDtypeStruct((_BATCH, _D), jnp.float32),
        jax.ShapeDtypeStruct((_BATCH, _D), jnp.float32),
        jax.ShapeDtypeStruct((_BATCH, _D), jnp.float32),
    ),
    mesh=_mesh,
    scratch_types=[
        pltpu.VMEM((_GPT,), jnp.int32),
        pltpu.VMEM((_GPT,), jnp.int32),
        pltpu.VMEM((_GPT,), jnp.int32),
        pltpu.VMEM((_GPT, _D), jnp.float32),
        pltpu.VMEM((_GPT, _D), jnp.float32),
        pltpu.VMEM((_GPT, _D), jnp.float32),
        pltpu.SemaphoreType.DMA,
    ],
)


def _loss_body(u_ref, p_ref, n_ref, o_ref):
    u = u_ref[...]
    diff = jnp.sum(u * (p_ref[...] - n_ref[...]), axis=1)
    o_ref[...] = jnp.reshape(-jnp.mean(jax.nn.log_sigmoid(diff)), (1, 1))


_loss = pl.pallas_call(
    _loss_body,
    out_shape=jax.ShapeDtypeStruct((1, 1), jnp.float32),
)


def _gather_view(x):
    xb = x.astype(jnp.bfloat16).reshape(-1, 4, 2, 16)
    xb = jnp.transpose(xb, (0, 1, 3, 2)).reshape(-1, 64, 2)
    return jax.lax.bitcast_convert_type(xb, jnp.float32)


def kernel(user_id, pos_item, neg_item, edge_index, edge_values, user_emb, item_emb):
    rows = edge_index[0].astype(jnp.int32).reshape(_NW, _NBLK, _EB)
    cols = edge_index[1].astype(jnp.int32).reshape(_NW, _NBLK, _EB)
    vals = edge_values

    e0 = jnp.concatenate([user_emb, item_emb], axis=0)

    cur = e0
    normed = []
    for layer in range(_LAYERS - 1):
        parts = _propagate(_gather_view(cur), rows, cols, vals)
        cur = _norm(parts)
        normed.append(cur)
    parts = _propagate(_gather_view(cur), rows, cols, vals)
    final_embedding = _final(parts, e0, normed[0], normed[1])

    ug, pg, ng = _gather(final_embedding,
                         user_id.astype(jnp.int32),
                         pos_item.astype(jnp.int32),
                         neg_item.astype(jnp.int32))
    rec_loss = _loss(ug, pg, ng)[0, 0]
    return (rec_loss, final_embedding)
